# baseline (speedup 1.0000x reference)
import jax
import jax.numpy as jnp
from jax.experimental import pallas as pl
from jax.experimental.pallas import tpu as pltpu

_N, _D, _K = 8192, 64, 1024
_BLK = 512
_NBLK = _N // _BLK
_MAX_ITERS = 40


def _iter_body(data_ref, means_ref, mnew_ref, delta_ref, sums_ref, counts_ref):
    i = pl.program_id(0)

    @pl.when(i == 0)
    def _init():
        sums_ref[...] = jnp.zeros_like(sums_ref)
        counts_ref[...] = jnp.zeros_like(counts_ref)

    x = data_ref[...]
    m = means_ref[...]
    x2 = jnp.sum(x * x, axis=1, keepdims=True)
    m2 = jnp.sum(m * m, axis=1)[None, :]
    xm = jax.lax.dot_general(x, m, (((1,), (1,)), ((), ())),
                             preferred_element_type=jnp.float32)
    d2 = x2 - 2.0 * xm + m2
    labels = jnp.argmin(d2, axis=1)
    onehot = (labels[:, None] ==
              jax.lax.broadcasted_iota(jnp.int32, (_BLK, _K), 1)
              ).astype(jnp.float32)
    sums_ref[...] += jax.lax.dot_general(
        onehot, x, (((0,), (0,)), ((), ())),
        preferred_element_type=jnp.float32,
        precision=jax.lax.Precision.HIGHEST)
    ones = jnp.ones((_BLK, 1), jnp.float32)
    counts_ref[...] += jax.lax.dot_general(
        onehot, ones, (((0,), (0,)), ((), ())),
        preferred_element_type=jnp.float32,
        precision=jax.lax.Precision.HIGHEST)

    @pl.when(i == _NBLK - 1)
    def _finalize():
        counts = counts_ref[...]
        m_old = means_ref[...]
        m_new = jnp.where(counts > 0.0,
                          sums_ref[...] / jnp.maximum(counts, 1.0),
                          m_old)
        mnew_ref[...] = m_new
        diff = m_old - m_new
        delta_ref[0, 0] = jnp.max(jnp.sum(diff * diff, axis=1))


def _lloyd_iter(Data, means):
    mnew, delta = pl.pallas_call(
        _iter_body,
        grid=(_NBLK,),
        in_specs=[
            pl.BlockSpec((_BLK, _D), lambda i: (i, 0)),
            pl.BlockSpec((_K, _D), lambda i: (0, 0)),
        ],
        out_specs=[
            pl.BlockSpec((_K, _D), lambda i: (0, 0)),
            pl.BlockSpec(memory_space=pltpu.SMEM),
        ],
        out_shape=[
            jax.ShapeDtypeStruct((_K, _D), jnp.float32),
            jax.ShapeDtypeStruct((1, 1), jnp.float32),
        ],
        scratch_shapes=[
            pltpu.VMEM((_K, _D), jnp.float32),
            pltpu.VMEM((_K, 1), jnp.float32),
        ],
    )(Data, means)
    return mnew, delta[0, 0]


def kernel(Data, means, threshold):
    threshold = jnp.asarray(threshold)
    delta0 = threshold.astype(jnp.float32) + 1.0
    it0 = jnp.asarray(0, dtype=jnp.int32)

    def cond_fun(carry):
        _, delta, it = carry
        return (delta > threshold) & (it < _MAX_ITERS)

    def body_fun(carry):
        means_cur, _, it = carry
        means_new, delta = _lloyd_iter(Data, means_cur)
        return means_new, delta, it + 1

    means_final, _, _ = jax.lax.while_loop(cond_fun, body_fun,
                                           (means, delta0, it0))
    return means_final


# speedup vs baseline: 1.0436x; 1.0436x over previous
import jax
import jax.numpy as jnp
from jax.experimental import pallas as pl
from jax.experimental.pallas import tpu as pltpu

_N, _D, _K = 8192, 64, 1024
_BLK = 512
_NBLK = _N // _BLK
_MAX_ITERS = 40


def _persistent_body(thr_ref, data_ref, means_ref, out_ref,
                     sums_ref, counts_ref, x2_ref):
    out_ref[...] = means_ref[...]
    data = data_ref[...]
    x2_ref[...] = jnp.sum(data * data, axis=1, keepdims=True)

    def one_iter():
        m = out_ref[...]
        m2 = jnp.sum(m * m, axis=1)[None, :]
        sums_ref[...] = jnp.zeros_like(sums_ref)
        counts_ref[...] = jnp.zeros_like(counts_ref)

        def blk(b, carry):
            x = data_ref[pl.ds(b * _BLK, _BLK), :]
            x2 = x2_ref[pl.ds(b * _BLK, _BLK), :]
            xm = jax.lax.dot_general(x, m, (((1,), (1,)), ((), ())),
                                     preferred_element_type=jnp.float32)
            d2 = x2 - 2.0 * xm + m2
            labels = jnp.argmin(d2, axis=1)
            onehot = (labels[:, None] ==
                      jax.lax.broadcasted_iota(jnp.int32, (_BLK, _K), 1)
                      ).astype(jnp.float32)
            sums_ref[...] += jax.lax.dot_general(
                onehot, x, (((0,), (0,)), ((), ())),
                preferred_element_type=jnp.float32,
                precision=jax.lax.Precision.HIGHEST)
            ones = jnp.ones((_BLK, 1), jnp.float32)
            counts_ref[...] += jax.lax.dot_general(
                onehot, ones, (((0,), (0,)), ((), ())),
                preferred_element_type=jnp.float32,
                precision=jax.lax.Precision.HIGHEST)
            return carry

        jax.lax.fori_loop(0, _NBLK, blk, 0)
        counts = counts_ref[...]
        m_new = jnp.where(counts > 0.0,
                          sums_ref[...] / jnp.maximum(counts, 1.0),
                          m)
        out_ref[...] = m_new
        diff = m - m_new
        return jnp.max(jnp.sum(diff * diff, axis=1))

    thr = thr_ref[0, 0]

    def cond_fun(carry):
        delta, it = carry
        return (delta > thr) & (it < _MAX_ITERS)

    def body_fun(carry):
        _, it = carry
        delta = one_iter()
        return delta, it + 1

    jax.lax.while_loop(cond_fun, body_fun,
                       (thr + 1.0, jnp.asarray(0, jnp.int32)))


def kernel(Data, means, threshold):
    thr = jnp.asarray(threshold, jnp.float32).reshape(1, 1)
    means_final = pl.pallas_call(
        _persistent_body,
        in_specs=[
            pl.BlockSpec(memory_space=pltpu.SMEM),
            pl.BlockSpec(memory_space=pltpu.VMEM),
            pl.BlockSpec(memory_space=pltpu.VMEM),
        ],
        out_specs=pl.BlockSpec(memory_space=pltpu.VMEM),
        out_shape=jax.ShapeDtypeStruct((_K, _D), jnp.float32),
        scratch_shapes=[
            pltpu.VMEM((_K, _D), jnp.float32),
            pltpu.VMEM((_K, 1), jnp.float32),
            pltpu.VMEM((_N, 1), jnp.float32),
        ],
    )(thr, Data, means)
    return means_final


# speedup vs baseline: 1.7497x; 1.6766x over previous
import jax
import jax.numpy as jnp
from jax.experimental import pallas as pl
from jax.experimental.pallas import tpu as pltpu

_N, _D, _K = 8192, 64, 1024
_BLK = 512
_NBLK = _N // _BLK
_MAX_ITERS = 40


def _persistent_body(thr_ref, data_ref, means_ref, out_ref,
                     sums_ref, counts_ref, x2_ref,
                     hi_ref, mid_ref, lo_ref):
    out_ref[...] = means_ref[...]
    data = data_ref[...]
    x2_ref[...] = jnp.sum(data * data, axis=1, keepdims=True)
    hi = data.astype(jnp.bfloat16)
    r1 = data - hi.astype(jnp.float32)
    mid = r1.astype(jnp.bfloat16)
    r2 = r1 - mid.astype(jnp.float32)
    hi_ref[...] = hi
    mid_ref[...] = mid
    lo_ref[...] = r2.astype(jnp.bfloat16)

    def one_iter():
        m = out_ref[...]
        m2 = jnp.sum(m * m, axis=1)[None, :]
        sums_ref[...] = jnp.zeros_like(sums_ref)
        counts_ref[...] = jnp.zeros_like(counts_ref)

        def blk(b, carry):
            x = data_ref[pl.ds(b * _BLK, _BLK), :]
            x2 = x2_ref[pl.ds(b * _BLK, _BLK), :]
            xm = jax.lax.dot_general(x, m, (((1,), (1,)), ((), ())),
                                     preferred_element_type=jnp.float32)
            d2 = x2 - 2.0 * xm + m2
            labels = jnp.argmin(d2, axis=1)
            onehot = (labels[:, None] ==
                      jax.lax.broadcasted_iota(jnp.int32, (_BLK, _K), 1)
                      ).astype(jnp.bfloat16)
            acc = jnp.zeros((_K, _D), jnp.float32)
            for part_ref in (hi_ref, mid_ref, lo_ref):
                p = part_ref[pl.ds(b * _BLK, _BLK), :]
                acc += jax.lax.dot_general(
                    onehot, p, (((0,), (0,)), ((), ())),
                    preferred_element_type=jnp.float32)
            sums_ref[...] += acc
            ones = jnp.ones((_BLK, 1), jnp.bfloat16)
            counts_ref[...] += jax.lax.dot_general(
                onehot, ones, (((0,), (0,)), ((), ())),
                preferred_element_type=jnp.float32)
            return carry

        jax.lax.fori_loop(0, _NBLK, blk, 0)
        counts = counts_ref[...]
        m_new = jnp.where(counts > 0.0,
                          sums_ref[...] / jnp.maximum(counts, 1.0),
                          m)
        out_ref[...] = m_new
        diff = m - m_new
        return jnp.max(jnp.sum(diff * diff, axis=1))

    thr = thr_ref[0, 0]

    def cond_fun(carry):
        delta, it = carry
        return (delta > thr) & (it < _MAX_ITERS)

    def body_fun(carry):
        _, it = carry
        delta = one_iter()
        return delta, it + 1

    jax.lax.while_loop(cond_fun, body_fun,
                       (thr + 1.0, jnp.asarray(0, jnp.int32)))


def kernel(Data, means, threshold):
    thr = jnp.asarray(threshold, jnp.float32).reshape(1, 1)
    means_final = pl.pallas_call(
        _persistent_body,
        in_specs=[
            pl.BlockSpec(memory_space=pltpu.SMEM),
            pl.BlockSpec(memory_space=pltpu.VMEM),
            pl.BlockSpec(memory_space=pltpu.VMEM),
        ],
        out_specs=pl.BlockSpec(memory_space=pltpu.VMEM),
        out_shape=jax.ShapeDtypeStruct((_K, _D), jnp.float32),
        scratch_shapes=[
            pltpu.VMEM((_K, _D), jnp.float32),
            pltpu.VMEM((_K, 1), jnp.float32),
            pltpu.VMEM((_N, 1), jnp.float32),
            pltpu.VMEM((_N, _D), jnp.bfloat16),
            pltpu.VMEM((_N, _D), jnp.bfloat16),
            pltpu.VMEM((_N, _D), jnp.bfloat16),
        ],
    )(thr, Data, means)
    return means_final


# speedup vs baseline: 3.1109x; 1.7780x over previous
import jax
import jax.numpy as jnp
from jax.experimental import pallas as pl
from jax.experimental.pallas import tpu as pltpu

_N, _D, _K = 8192, 64, 1024
_BLK = 512
_NBLK = _N // _BLK
_MAX_ITERS = 40
_W = 4 * _D


def _persistent_body(thr_ref, data_ref, means_ref, out_ref,
                     acc_ref, x2_ref, hml_ref):
    out_ref[...] = means_ref[...]
    data = data_ref[...]
    x2_ref[...] = jnp.sum(data * data, axis=1, keepdims=True)
    hi = data.astype(jnp.bfloat16)
    r1 = data - hi.astype(jnp.float32)
    mid = r1.astype(jnp.bfloat16)
    r2 = r1 - mid.astype(jnp.float32)
    hml_ref[:, 0 * _D:1 * _D] = hi
    hml_ref[:, 1 * _D:2 * _D] = mid
    hml_ref[:, 2 * _D:3 * _D] = r2.astype(jnp.bfloat16)
    hml_ref[:, 3 * _D:4 * _D] = jnp.ones((_N, _D), jnp.bfloat16)

    def one_iter():
        m = out_ref[...]
        m2 = jnp.sum(m * m, axis=1)[None, :]
        acc_ref[...] = jnp.zeros_like(acc_ref)

        def blk(b, carry):
            x = data_ref[pl.ds(b * _BLK, _BLK), :]
            x2 = x2_ref[pl.ds(b * _BLK, _BLK), :]
            xm = jax.lax.dot_general(x, m, (((1,), (1,)), ((), ())),
                                     preferred_element_type=jnp.float32)
            d2 = x2 - 2.0 * xm + m2
            labels = jnp.argmin(d2, axis=1)
            onehot = (labels[:, None] ==
                      jax.lax.broadcasted_iota(jnp.int32, (_BLK, _K), 1)
                      ).astype(jnp.bfloat16)
            hml = hml_ref[pl.ds(b * _BLK, _BLK), :]
            acc_ref[...] += jax.lax.dot_general(
                onehot, hml, (((0,), (0,)), ((), ())),
                preferred_element_type=jnp.float32)
            return carry

        jax.lax.fori_loop(0, _NBLK, blk, 0)
        acc = acc_ref[...]
        sums = (acc[:, 0 * _D:1 * _D] + acc[:, 1 * _D:2 * _D]
                + acc[:, 2 * _D:3 * _D])
        counts = acc[:, 3 * _D:3 * _D + 1]
        m_new = jnp.where(counts > 0.0,
                          sums / jnp.maximum(counts, 1.0),
                          m)
        out_ref[...] = m_new
        diff = m - m_new
        return jnp.max(jnp.sum(diff * diff, axis=1))

    thr = thr_ref[0, 0]

    def cond_fun(carry):
        delta, it = carry
        return (delta > thr) & (it < _MAX_ITERS)

    def body_fun(carry):
        _, it = carry
        delta = one_iter()
        return delta, it + 1

    jax.lax.while_loop(cond_fun, body_fun,
                       (thr + 1.0, jnp.asarray(0, jnp.int32)))


def kernel(Data, means, threshold):
    thr = jnp.asarray(threshold, jnp.float32).reshape(1, 1)
    means_final = pl.pallas_call(
        _persistent_body,
        in_specs=[
            pl.BlockSpec(memory_space=pltpu.SMEM),
            pl.BlockSpec(memory_space=pltpu.VMEM),
            pl.BlockSpec(memory_space=pltpu.VMEM),
        ],
        out_specs=pl.BlockSpec(memory_space=pltpu.VMEM),
        out_shape=jax.ShapeDtypeStruct((_K, _D), jnp.float32),
        scratch_shapes=[
            pltpu.VMEM((_K, _W), jnp.float32),
            pltpu.VMEM((_N, 1), jnp.float32),
            pltpu.VMEM((_N, _W), jnp.bfloat16),
        ],
    )(thr, Data, means)
    return means_final
